# Initial kernel scaffold; baseline (speedup 1.0000x reference)
import jax
import jax.numpy as jnp
from jax.experimental import pallas as pl


def kernel(entity_emb, relation_emb, edge_index, edge_type, W_fc, b_fc):
    raise NotImplementedError("write your pallas kernel here")


# speedup vs baseline: 5.8629x; 5.8629x over previous
import functools

import jax
import jax.numpy as jnp
from jax import lax
from jax.experimental import pallas as pl
from jax.experimental.pallas import tpu as pltpu
from jax.experimental.pallas import tpu_sc as plsc

N = 10000
E = 320000
D = 128
R = 64
NW = 32
NPT = 320
NPAD = NW * NPT
AW = D + 16
G = 128
EPAD = E + G


def _prep_body(emb_ref, rel_ref, w_ref, b_ref, a_ref, b2_ref, aug_ref):
    rel = rel_ref[...]
    w = w_ref[...]
    p = lax.dot_general(rel, w, (((1,), (0,)), ((), ())),
                        preferred_element_type=jnp.float32)
    emb = emb_ref[...]
    a = lax.dot_general(emb, p[:, :D], (((1,), (1,)), ((), ())),
                        preferred_element_type=jnp.float32)
    bmat = lax.dot_general(emb, p[:, D:], (((1,), (1,)), ((), ())),
                           preferred_element_type=jnp.float32)
    c = lax.dot_general(rel, b_ref[0:1, :], (((1,), (1,)), ((), ())),
                        preferred_element_type=jnp.float32)
    a_ref[...] = a + jnp.reshape(c, (1, R))
    b2_ref[...] = bmat
    aug_ref[...] = jnp.concatenate(
        [emb, jnp.ones((emb.shape[0], 16), jnp.float32)], axis=1)


def _prep(emb, rel, w_fc, b_pad):
    bn = 2000
    grid = N // bn
    return pl.pallas_call(
        _prep_body,
        grid=(grid,),
        in_specs=[
            pl.BlockSpec((bn, D), lambda i: (i, 0)),
            pl.BlockSpec((R, D), lambda i: (0, 0)),
            pl.BlockSpec((D, 2 * D), lambda i: (0, 0)),
            pl.BlockSpec((8, D), lambda i: (0, 0)),
        ],
        out_specs=[
            pl.BlockSpec((bn, R), lambda i: (i, 0)),
            pl.BlockSpec((bn, R), lambda i: (i, 0)),
            pl.BlockSpec((bn, AW), lambda i: (i, 0)),
        ],
        out_shape=[
            jax.ShapeDtypeStruct((N, R), jnp.float32),
            jax.ShapeDtypeStruct((N, R), jnp.float32),
            jax.ShapeDtypeStruct((N, AW), jnp.float32),
        ],
    )(emb, rel, w_fc, b_pad)


def _edge_body(head_hbm, tail_hbm, et_hbm, a_hbm, b_hbm, aug_hbm, offs_hbm,
               out_hbm, headg, tailg, etg, ia, ib, maskf,
               avals, bvals, rows, sv, acc,
               offs_sh, stage_i_sh, stage_f_sh,
               offs_sm, hs_sm, ss_sm, sema, semb, semr):
    cid = lax.axis_index("c")
    sid = lax.axis_index("s")
    wid = cid * 16 + sid
    n0 = wid * NPT

    def zero_body(i, carry):
        for d in range(AW // 16):
            acc[i, pl.ds(d * 16, 16)] = jnp.zeros((16,), jnp.float32)
        return carry

    lax.fori_loop(0, NPT, zero_body, 0)

    pltpu.sync_copy(offs_hbm, offs_sh.at[sid])
    pltpu.sync_copy(offs_sh.at[sid], offs_sm)
    off_lo = offs_sm[wid]
    off_hi = offs_sm[wid + 1]
    base0 = (off_lo >> 3) << 3
    ngroups = (off_hi - base0 + (G - 1)) >> 7

    n0v = jnp.full((16,), n0, jnp.int32)
    n1v = jnp.full((16,), n0 + NPT, jnp.int32)
    ev_lim = jnp.full((16,), E, jnp.int32)
    i16 = lax.broadcasted_iota(jnp.int32, (16,), 0)

    def grp_body(g, carry):
        base = pl.multiple_of(base0 + g * G, 8)
        pltpu.sync_copy(head_hbm.at[pl.ds(base, G)], headg)
        pltpu.sync_copy(tail_hbm.at[pl.ds(base, G)], tailg)
        pltpu.sync_copy(et_hbm.at[pl.ds(base, G)], etg)
        for k8 in range(G // 16):
            sl = pl.ds(k8 * 16, 16)
            hv = headg[sl]
            tv = tailg[sl]
            ev = etg[sl]
            ia[sl] = hv * R + ev
            ib[sl] = tv * R + ev
            eidx = jnp.full((16,), base + k8 * 16, jnp.int32) + i16
            ok = (hv >= n0v) & (hv < n1v) & (eidx < ev_lim)
            maskf[sl] = jnp.where(ok, jnp.float32(1.0), jnp.float32(0.0))
        ca = pltpu.async_copy(a_hbm.at[ia], avals, sema)
        cb = pltpu.async_copy(b_hbm.at[ib], bvals, semb)
        cr = pltpu.async_copy(aug_hbm.at[tailg], rows, semr)
        pltpu.sync_copy(headg, stage_i_sh.at[sid])
        pltpu.sync_copy(stage_i_sh.at[sid], hs_sm)
        ca.wait()
        cb.wait()
        cr.wait()
        for k8 in range(G // 16):
            sl = pl.ds(k8 * 16, 16)
            x = avals[sl] + bvals[sl]
            x = jnp.maximum(x, x * jnp.float32(0.2))
            sv[sl] = jnp.exp(x) * maskf[sl]
        pltpu.sync_copy(sv, stage_f_sh.at[sid])
        pltpu.sync_copy(stage_f_sh.at[sid], ss_sm)

        def edge_one(k, inner):
            lid = jnp.minimum(jnp.maximum(hs_sm[k] - n0, 0), NPT - 1)
            s_k = ss_sm[k]
            for d in range(AW // 16):
                dsl = pl.ds(d * 16, 16)
                acc[lid, dsl] += rows[k, dsl] * s_k
            return inner

        lax.fori_loop(0, G, edge_one, 0)
        return carry

    lax.fori_loop(0, ngroups, grp_body, 0)
    pltpu.sync_copy(acc, out_hbm.at[pl.ds(n0, NPT)])


_EDGE_SCRATCH = [
        pltpu.VMEM((G,), jnp.int32),
        pltpu.VMEM((G,), jnp.int32),
        pltpu.VMEM((G,), jnp.int32),
        pltpu.VMEM((G,), jnp.int32),
        pltpu.VMEM((G,), jnp.int32),
        pltpu.VMEM((G,), jnp.float32),
        pltpu.VMEM((G,), jnp.float32),
        pltpu.VMEM((G,), jnp.float32),
        pltpu.VMEM((G, AW), jnp.float32),
        pltpu.VMEM((G,), jnp.float32),
        pltpu.VMEM((NPT, AW), jnp.float32),
        pltpu.VMEM_SHARED((16, 48), jnp.int32),
        pltpu.VMEM_SHARED((16, G), jnp.int32),
        pltpu.VMEM_SHARED((16, G), jnp.float32),
        pltpu.SMEM((48,), jnp.int32),
        pltpu.SMEM((G,), jnp.int32),
        pltpu.SMEM((G,), jnp.float32),
        pltpu.SemaphoreType.DMA,
        pltpu.SemaphoreType.DMA,
        pltpu.SemaphoreType.DMA,
]

_edge_pass = functools.partial(
    pl.kernel,
    compiler_params=pltpu.CompilerParams(use_tc_tiling_on_sc=False),
    out_type=jax.ShapeDtypeStruct((NPAD, AW), jnp.float32),
    mesh=plsc.VectorSubcoreMesh(core_axis_name="c", subcore_axis_name="s"),
    scratch_types=_EDGE_SCRATCH,
)(_edge_body)


def _node_body(acc_ref, emb_ref, out_ref):
    agg = acc_ref[:, :D]
    den = acc_ref[:, D:D + 1]
    mid = agg / (den + jnp.float32(1e-16)) + emb_ref[...]
    nrm = jnp.sqrt(jnp.sum(mid * mid, axis=1, keepdims=True))
    out_ref[...] = mid / jnp.maximum(nrm, jnp.float32(1e-12))


def _node_update(acc, emb):
    bn = 2000
    return pl.pallas_call(
        _node_body,
        grid=(N // bn,),
        in_specs=[
            pl.BlockSpec((bn, AW), lambda i: (i, 0)),
            pl.BlockSpec((bn, D), lambda i: (i, 0)),
        ],
        out_specs=pl.BlockSpec((bn, D), lambda i: (i, 0)),
        out_shape=jax.ShapeDtypeStruct((N, D), jnp.float32),
    )(acc, emb)


def kernel(entity_emb, relation_emb, edge_index, edge_type, W_fc, b_fc):
    head = edge_index[0].astype(jnp.int32)
    tail = edge_index[1].astype(jnp.int32)
    et = edge_type.astype(jnp.int32)

    order = jnp.argsort(head)
    head_s = head[order]
    tail_s = tail[order]
    et_s = et[order]
    pad = EPAD - E
    head_p = jnp.concatenate([head_s, jnp.zeros((pad,), jnp.int32)])
    tail_p = jnp.concatenate([tail_s, jnp.zeros((pad,), jnp.int32)])
    et_p = jnp.concatenate([et_s, jnp.zeros((pad,), jnp.int32)])
    bounds = (jnp.arange(33, dtype=jnp.int32) * NPT)
    offs = jnp.searchsorted(head_s, bounds).astype(jnp.int32)
    offs = jnp.concatenate([offs, jnp.full((15,), E, jnp.int32)])

    b_pad = jnp.zeros((8, D), jnp.float32).at[0].set(b_fc)

    emb = entity_emb
    for _ in range(2):
        a_tab, b_tab, aug = _prep(emb, relation_emb, W_fc, b_pad)
        acc = _edge_pass(head_p, tail_p, et_p,
                         a_tab.reshape(N * R), b_tab.reshape(N * R),
                         aug, offs)
        emb = _node_update(acc[:N], emb)
    return emb


# speedup vs baseline: 6.8165x; 1.1627x over previous
import functools

import jax
import jax.numpy as jnp
from jax import lax
from jax.experimental import pallas as pl
from jax.experimental.pallas import tpu as pltpu
from jax.experimental.pallas import tpu_sc as plsc

N = 10000
E = 320000
D = 128
R = 64
NW = 32
NPT = 320
NPAD = NW * NPT
AW = D + 16
G = 128
EPAD = E + 4 * G


def _prep_body(emb_ref, rel_ref, w_ref, b_ref, a_ref, b2_ref, aug_ref):
    rel = rel_ref[...]
    w = w_ref[...]
    p = lax.dot_general(rel, w, (((1,), (0,)), ((), ())),
                        preferred_element_type=jnp.float32)
    emb = emb_ref[...]
    a = lax.dot_general(emb, p[:, :D], (((1,), (1,)), ((), ())),
                        preferred_element_type=jnp.float32)
    bmat = lax.dot_general(emb, p[:, D:], (((1,), (1,)), ((), ())),
                           preferred_element_type=jnp.float32)
    c = lax.dot_general(rel, b_ref[0:1, :], (((1,), (1,)), ((), ())),
                        preferred_element_type=jnp.float32)
    a_ref[...] = a + jnp.reshape(c, (1, R))
    b2_ref[...] = bmat
    aug_ref[...] = jnp.concatenate(
        [emb, jnp.ones((emb.shape[0], 16), jnp.float32)], axis=1)


def _prep(emb, rel, w_fc, b_pad):
    bn = 2000
    grid = N // bn
    return pl.pallas_call(
        _prep_body,
        grid=(grid,),
        in_specs=[
            pl.BlockSpec((bn, D), lambda i: (i, 0)),
            pl.BlockSpec((R, D), lambda i: (0, 0)),
            pl.BlockSpec((D, 2 * D), lambda i: (0, 0)),
            pl.BlockSpec((8, D), lambda i: (0, 0)),
        ],
        out_specs=[
            pl.BlockSpec((bn, R), lambda i: (i, 0)),
            pl.BlockSpec((bn, R), lambda i: (i, 0)),
            pl.BlockSpec((bn, AW), lambda i: (i, 0)),
        ],
        out_shape=[
            jax.ShapeDtypeStruct((N, R), jnp.float32),
            jax.ShapeDtypeStruct((N, R), jnp.float32),
            jax.ShapeDtypeStruct((N, AW), jnp.float32),
        ],
    )(emb, rel, w_fc, b_pad)


def _edge_body(edges_hbm, a_hbm, b_hbm, aug_hbm, offs_hbm, out_hbm,
               edg0, edg1, ia0, ia1, ib0, ib1, mk0, mk1,
               av0, av1, bv0, bv1, rw0, rw1, sv0, sv1, acc,
               offs_sh, sti, stf, offs_sm, hs0, hs1, ss0, ss1,
               sa0, sa1, sb0, sb1, sr0, sr1):
    edg = (edg0, edg1)
    ia = (ia0, ia1)
    ib = (ib0, ib1)
    mk = (mk0, mk1)
    av = (av0, av1)
    bv = (bv0, bv1)
    rw = (rw0, rw1)
    sv = (sv0, sv1)
    hs = (hs0, hs1)
    ss = (ss0, ss1)
    sa = (sa0, sa1)
    sb = (sb0, sb1)
    sr = (sr0, sr1)

    cid = lax.axis_index("c")
    sid = lax.axis_index("s")
    wid = cid * 16 + sid
    n0 = wid * NPT

    def zero_body(i, carry):
        for d in range(AW // 16):
            acc[i, pl.ds(d * 16, 16)] = jnp.zeros((16,), jnp.float32)
        return carry

    lax.fori_loop(0, NPT, zero_body, 0)

    pltpu.sync_copy(offs_hbm, offs_sh.at[sid])
    pltpu.sync_copy(offs_sh.at[sid], offs_sm)
    off_lo = offs_sm[wid]
    off_hi = offs_sm[wid + 1]
    base0 = (off_lo >> 3) << 3
    ngroups = (off_hi - base0 + (G - 1)) >> 7
    npairs = (ngroups + 1) >> 1

    n0v = jnp.full((16,), n0, jnp.int32)
    n1v = jnp.full((16,), n0 + NPT, jnp.int32)
    ev_lim = jnp.full((16,), E, jnp.int32)
    i16 = lax.broadcasted_iota(jnp.int32, (16,), 0)

    def load_prep_fire(g, b):
        base = pl.multiple_of(base0 + g * G, 8)
        pltpu.sync_copy(edges_hbm.at[:, pl.ds(base, G)], edg[b])
        for k8 in range(G // 16):
            sl = pl.ds(k8 * 16, 16)
            hv = edg[b][0, sl]
            tv = edg[b][1, sl]
            ev = edg[b][2, sl]
            ia[b][sl] = hv * R + ev
            ib[b][sl] = tv * R + ev
            eidx = jnp.full((16,), base + k8 * 16, jnp.int32) + i16
            ok = (hv >= n0v) & (hv < n1v) & (eidx < ev_lim)
            mk[b][sl] = jnp.where(ok, jnp.float32(1.0), jnp.float32(0.0))
        ca = pltpu.async_copy(a_hbm.at[ia[b]], av[b], sa[b])
        cb = pltpu.async_copy(b_hbm.at[ib[b]], bv[b], sb[b])
        cr = pltpu.async_copy(aug_hbm.at[edg[b].at[1]], rw[b], sr[b])
        pltpu.sync_copy(edg[b].at[0], sti.at[sid])
        pltpu.sync_copy(sti.at[sid], hs[b])
        return ca, cb, cr

    def wait_set(b):
        pltpu.make_async_copy(a_hbm.at[ia[b]], av[b], sa[b]).wait()
        pltpu.make_async_copy(b_hbm.at[ib[b]], bv[b], sb[b]).wait()
        pltpu.make_async_copy(aug_hbm.at[edg[b].at[1]], rw[b], sr[b]).wait()

    load_prep_fire(0, 0)

    def pair_body(gp, carry):
        for b in range(2):
            g = 2 * gp + b
            load_prep_fire(g + 1, 1 - b)
            wait_set(b)
            for k8 in range(G // 16):
                sl = pl.ds(k8 * 16, 16)
                x = av[b][sl] + bv[b][sl]
                x = jnp.maximum(x, x * jnp.float32(0.2))
                sv[b][sl] = jnp.exp(x) * mk[b][sl]
            pltpu.sync_copy(sv[b], stf.at[sid])
            pltpu.sync_copy(stf.at[sid], ss[b])

            def edge_one(k, inner):
                lid = jnp.minimum(jnp.maximum(hs[b][k] - n0, 0), NPT - 1)
                s_k = ss[b][k]
                for d in range(AW // 16):
                    dsl = pl.ds(d * 16, 16)
                    acc[lid, dsl] += rw[b][k, dsl] * s_k
                return inner

            lax.fori_loop(0, G, edge_one, 0)
        return carry

    lax.fori_loop(0, npairs, pair_body, 0)
    wait_set(0)
    pltpu.sync_copy(acc, out_hbm.at[pl.ds(n0, NPT)])


_EDGE_SCRATCH = [
    pltpu.VMEM((4, G), jnp.int32),
    pltpu.VMEM((4, G), jnp.int32),
    pltpu.VMEM((G,), jnp.int32),
    pltpu.VMEM((G,), jnp.int32),
    pltpu.VMEM((G,), jnp.int32),
    pltpu.VMEM((G,), jnp.int32),
    pltpu.VMEM((G,), jnp.float32),
    pltpu.VMEM((G,), jnp.float32),
    pltpu.VMEM((G,), jnp.float32),
    pltpu.VMEM((G,), jnp.float32),
    pltpu.VMEM((G,), jnp.float32),
    pltpu.VMEM((G,), jnp.float32),
    pltpu.VMEM((G, AW), jnp.float32),
    pltpu.VMEM((G, AW), jnp.float32),
    pltpu.VMEM((G,), jnp.float32),
    pltpu.VMEM((G,), jnp.float32),
    pltpu.VMEM((NPT, AW), jnp.float32),
    pltpu.VMEM_SHARED((16, 48), jnp.int32),
    pltpu.VMEM_SHARED((16, G), jnp.int32),
    pltpu.VMEM_SHARED((16, G), jnp.float32),
    pltpu.SMEM((48,), jnp.int32),
    pltpu.SMEM((G,), jnp.int32),
    pltpu.SMEM((G,), jnp.int32),
    pltpu.SMEM((G,), jnp.float32),
    pltpu.SMEM((G,), jnp.float32),
    pltpu.SemaphoreType.DMA,
    pltpu.SemaphoreType.DMA,
    pltpu.SemaphoreType.DMA,
    pltpu.SemaphoreType.DMA,
    pltpu.SemaphoreType.DMA,
    pltpu.SemaphoreType.DMA,
]

_edge_pass = functools.partial(
    pl.kernel,
    compiler_params=pltpu.CompilerParams(use_tc_tiling_on_sc=False),
    out_type=jax.ShapeDtypeStruct((NPAD, AW), jnp.float32),
    mesh=plsc.VectorSubcoreMesh(core_axis_name="c", subcore_axis_name="s"),
    scratch_types=_EDGE_SCRATCH,
)(_edge_body)


def _node_body(acc_ref, emb_ref, out_ref):
    agg = acc_ref[:, :D]
    den = acc_ref[:, D:D + 1]
    mid = agg / (den + jnp.float32(1e-16)) + emb_ref[...]
    nrm = jnp.sqrt(jnp.sum(mid * mid, axis=1, keepdims=True))
    out_ref[...] = mid / jnp.maximum(nrm, jnp.float32(1e-12))


def _node_update(acc, emb):
    bn = 2000
    return pl.pallas_call(
        _node_body,
        grid=(N // bn,),
        in_specs=[
            pl.BlockSpec((bn, AW), lambda i: (i, 0)),
            pl.BlockSpec((bn, D), lambda i: (i, 0)),
        ],
        out_specs=pl.BlockSpec((bn, D), lambda i: (i, 0)),
        out_shape=jax.ShapeDtypeStruct((N, D), jnp.float32),
    )(acc, emb)


def kernel(entity_emb, relation_emb, edge_index, edge_type, W_fc, b_fc):
    head = edge_index[0].astype(jnp.int32)
    tail = edge_index[1].astype(jnp.int32)
    et = edge_type.astype(jnp.int32)

    order = jnp.argsort(head)
    head_s = head[order]
    tail_s = tail[order]
    et_s = et[order]
    pad = EPAD - E
    head_p = jnp.concatenate([head_s, jnp.zeros((pad,), jnp.int32)])
    tail_p = jnp.concatenate([tail_s, jnp.zeros((pad,), jnp.int32)])
    et_p = jnp.concatenate([et_s, jnp.zeros((pad,), jnp.int32)])
    edges_p = jnp.stack([head_p, tail_p, et_p, jnp.zeros((EPAD,), jnp.int32)])
    bounds = (jnp.arange(33, dtype=jnp.int32) * NPT)
    offs = jnp.searchsorted(head_s, bounds).astype(jnp.int32)
    offs = jnp.concatenate([offs, jnp.full((15,), E, jnp.int32)])

    b_pad = jnp.zeros((8, D), jnp.float32).at[0].set(b_fc)

    emb = entity_emb
    for _ in range(2):
        a_tab, b_tab, aug = _prep(emb, relation_emb, W_fc, b_pad)
        acc = _edge_pass(edges_p,
                         a_tab.reshape(N * R), b_tab.reshape(N * R),
                         aug, offs)
        emb = _node_update(acc[:N], emb)
    return emb


# speedup vs baseline: 7.6190x; 1.1177x over previous
import functools

import jax
import jax.numpy as jnp
from jax import lax
from jax.experimental import pallas as pl
from jax.experimental.pallas import tpu as pltpu
from jax.experimental.pallas import tpu_sc as plsc

N = 10000
E = 320000
D = 128
R = 64
NW = 32
NPT = 320
NPAD = NW * NPT
AW = D + 16
G = 128
EPAD = E + 4 * G


def _prep_body(emb_ref, rel_ref, w_ref, b_ref, a_ref, b2_ref, aug_ref):
    rel = rel_ref[...]
    w = w_ref[...]
    p = lax.dot_general(rel, w, (((1,), (0,)), ((), ())),
                        preferred_element_type=jnp.float32)
    emb = emb_ref[...]
    a = lax.dot_general(emb, p[:, :D], (((1,), (1,)), ((), ())),
                        preferred_element_type=jnp.float32)
    bmat = lax.dot_general(emb, p[:, D:], (((1,), (1,)), ((), ())),
                           preferred_element_type=jnp.float32)
    c = lax.dot_general(rel, b_ref[0:1, :], (((1,), (1,)), ((), ())),
                        preferred_element_type=jnp.float32)
    a_ref[...] = a + jnp.reshape(c, (1, R))
    b2_ref[...] = bmat
    aug_ref[...] = jnp.concatenate(
        [emb, jnp.ones((emb.shape[0], 16), jnp.float32)], axis=1)


def _prep(emb, rel, w_fc, b_pad):
    bn = 2000
    grid = N // bn
    return pl.pallas_call(
        _prep_body,
        grid=(grid,),
        in_specs=[
            pl.BlockSpec((bn, D), lambda i: (i, 0)),
            pl.BlockSpec((R, D), lambda i: (0, 0)),
            pl.BlockSpec((D, 2 * D), lambda i: (0, 0)),
            pl.BlockSpec((8, D), lambda i: (0, 0)),
        ],
        out_specs=[
            pl.BlockSpec((bn, R), lambda i: (i, 0)),
            pl.BlockSpec((bn, R), lambda i: (i, 0)),
            pl.BlockSpec((bn, AW), lambda i: (i, 0)),
        ],
        out_shape=[
            jax.ShapeDtypeStruct((N, R), jnp.float32),
            jax.ShapeDtypeStruct((N, R), jnp.float32),
            jax.ShapeDtypeStruct((N, AW), jnp.float32),
        ],
    )(emb, rel, w_fc, b_pad)


def _edge_body(edges_hbm, a_hbm, b_hbm, aug_hbm, offs_hbm, out_hbm,
               edg0, edg1, ia0, ia1, ib0, ib1, mk0, mk1,
               av0, av1, bv0, bv1, rw0, rw1, sv0, sv1, acc,
               offs_sh, sti, stf, offs_sm, hs0, hs1, ss0, ss1,
               sa0, sa1, sb0, sb1, sr0, sr1):
    edg = (edg0, edg1)
    ia = (ia0, ia1)
    ib = (ib0, ib1)
    mk = (mk0, mk1)
    av = (av0, av1)
    bv = (bv0, bv1)
    rw = (rw0, rw1)
    sv = (sv0, sv1)
    hs = (hs0, hs1)
    ss = (ss0, ss1)
    sa = (sa0, sa1)
    sb = (sb0, sb1)
    sr = (sr0, sr1)

    cid = lax.axis_index("c")
    sid = lax.axis_index("s")
    wid = cid * 16 + sid
    n0 = wid * NPT

    def zero_body(i, carry):
        for d in range(AW // 16):
            acc[i, pl.ds(d * 16, 16)] = jnp.zeros((16,), jnp.float32)
        return carry

    lax.fori_loop(0, NPT, zero_body, 0)

    pltpu.sync_copy(offs_hbm, offs_sh.at[sid])
    pltpu.sync_copy(offs_sh.at[sid], offs_sm)
    off_lo = offs_sm[wid]
    off_hi = offs_sm[wid + 1]
    base0 = (off_lo >> 3) << 3
    ngroups = (off_hi - base0 + (G - 1)) >> 7
    npairs = (ngroups + 1) >> 1

    n0v = jnp.full((16,), n0, jnp.int32)
    n1v = jnp.full((16,), n0 + NPT, jnp.int32)
    ev_lim = jnp.full((16,), E, jnp.int32)
    i16 = lax.broadcasted_iota(jnp.int32, (16,), 0)

    def load_prep_fire(g, b):
        base = pl.multiple_of(base0 + g * G, 8)
        pltpu.sync_copy(edges_hbm.at[:, pl.ds(base, G)], edg[b])
        for k8 in range(G // 16):
            sl = pl.ds(k8 * 16, 16)
            hv = edg[b][0, sl]
            tv = edg[b][1, sl]
            ev = edg[b][2, sl]
            ia[b][sl] = hv * R + ev
            ib[b][sl] = tv * R + ev
            eidx = jnp.full((16,), base + k8 * 16, jnp.int32) + i16
            ok = (hv >= n0v) & (hv < n1v) & (eidx < ev_lim)
            mk[b][sl] = jnp.where(ok, jnp.float32(1.0), jnp.float32(0.0))
        ca = pltpu.async_copy(a_hbm.at[ia[b]], av[b], sa[b])
        cb = pltpu.async_copy(b_hbm.at[ib[b]], bv[b], sb[b])
        cr = pltpu.async_copy(aug_hbm.at[edg[b].at[1]], rw[b], sr[b])
        pltpu.sync_copy(edg[b].at[0], sti.at[sid])
        pltpu.sync_copy(sti.at[sid], hs[b])
        return ca, cb, cr

    def wait_set(b):
        pltpu.make_async_copy(a_hbm.at[ia[b]], av[b], sa[b]).wait()
        pltpu.make_async_copy(b_hbm.at[ib[b]], bv[b], sb[b]).wait()
        pltpu.make_async_copy(aug_hbm.at[edg[b].at[1]], rw[b], sr[b]).wait()

    load_prep_fire(0, 0)

    def pair_body(gp, carry):
        for b in range(2):
            g = 2 * gp + b
            load_prep_fire(g + 1, 1 - b)
            wait_set(b)
            for k8 in range(G // 16):
                sl = pl.ds(k8 * 16, 16)
                x = av[b][sl] + bv[b][sl]
                x = jnp.maximum(x, x * jnp.float32(0.2))
                sv[b][sl] = jnp.exp(x) * mk[b][sl]
            pltpu.sync_copy(sv[b], stf.at[sid])
            pltpu.sync_copy(stf.at[sid], ss[b])

            def edge_two(k2, inner):
                for u in range(2):
                    k = k2 * 2 + u
                    lid = jnp.minimum(jnp.maximum(hs[b][k] - n0, 0), NPT - 1)
                    s_k = ss[b][k]
                    for d in range(AW // 16):
                        dsl = pl.ds(d * 16, 16)
                        plsc.addupdate(acc.at[lid, dsl], rw[b][k, dsl] * s_k)
                return inner

            lax.fori_loop(0, G // 2, edge_two, 0)
        return carry

    lax.fori_loop(0, npairs, pair_body, 0)
    wait_set(0)
    pltpu.sync_copy(acc, out_hbm.at[pl.ds(n0, NPT)])


_EDGE_SCRATCH = [
    pltpu.VMEM((4, G), jnp.int32),
    pltpu.VMEM((4, G), jnp.int32),
    pltpu.VMEM((G,), jnp.int32),
    pltpu.VMEM((G,), jnp.int32),
    pltpu.VMEM((G,), jnp.int32),
    pltpu.VMEM((G,), jnp.int32),
    pltpu.VMEM((G,), jnp.float32),
    pltpu.VMEM((G,), jnp.float32),
    pltpu.VMEM((G,), jnp.float32),
    pltpu.VMEM((G,), jnp.float32),
    pltpu.VMEM((G,), jnp.float32),
    pltpu.VMEM((G,), jnp.float32),
    pltpu.VMEM((G, AW), jnp.float32),
    pltpu.VMEM((G, AW), jnp.float32),
    pltpu.VMEM((G,), jnp.float32),
    pltpu.VMEM((G,), jnp.float32),
    pltpu.VMEM((NPT, AW), jnp.float32),
    pltpu.VMEM_SHARED((16, 48), jnp.int32),
    pltpu.VMEM_SHARED((16, G), jnp.int32),
    pltpu.VMEM_SHARED((16, G), jnp.float32),
    pltpu.SMEM((48,), jnp.int32),
    pltpu.SMEM((G,), jnp.int32),
    pltpu.SMEM((G,), jnp.int32),
    pltpu.SMEM((G,), jnp.float32),
    pltpu.SMEM((G,), jnp.float32),
    pltpu.SemaphoreType.DMA,
    pltpu.SemaphoreType.DMA,
    pltpu.SemaphoreType.DMA,
    pltpu.SemaphoreType.DMA,
    pltpu.SemaphoreType.DMA,
    pltpu.SemaphoreType.DMA,
]

_edge_pass = functools.partial(
    pl.kernel,
    compiler_params=pltpu.CompilerParams(use_tc_tiling_on_sc=False),
    out_type=jax.ShapeDtypeStruct((NPAD, AW), jnp.float32),
    mesh=plsc.VectorSubcoreMesh(core_axis_name="c", subcore_axis_name="s"),
    scratch_types=_EDGE_SCRATCH,
)(_edge_body)


def _node_body(acc_ref, emb_ref, out_ref):
    agg = acc_ref[:, :D]
    den = acc_ref[:, D:D + 1]
    mid = agg / (den + jnp.float32(1e-16)) + emb_ref[...]
    nrm = jnp.sqrt(jnp.sum(mid * mid, axis=1, keepdims=True))
    out_ref[...] = mid / jnp.maximum(nrm, jnp.float32(1e-12))


def _node_update(acc, emb):
    bn = 2000
    return pl.pallas_call(
        _node_body,
        grid=(N // bn,),
        in_specs=[
            pl.BlockSpec((bn, AW), lambda i: (i, 0)),
            pl.BlockSpec((bn, D), lambda i: (i, 0)),
        ],
        out_specs=pl.BlockSpec((bn, D), lambda i: (i, 0)),
        out_shape=jax.ShapeDtypeStruct((N, D), jnp.float32),
    )(acc, emb)


def kernel(entity_emb, relation_emb, edge_index, edge_type, W_fc, b_fc):
    head = edge_index[0].astype(jnp.int32)
    tail = edge_index[1].astype(jnp.int32)
    et = edge_type.astype(jnp.int32)

    head_s, tail_s, et_s = lax.sort((head, tail, et), dimension=0, num_keys=1)
    pad = EPAD - E
    head_p = jnp.concatenate([head_s, jnp.zeros((pad,), jnp.int32)])
    tail_p = jnp.concatenate([tail_s, jnp.zeros((pad,), jnp.int32)])
    et_p = jnp.concatenate([et_s, jnp.zeros((pad,), jnp.int32)])
    edges_p = jnp.stack([head_p, tail_p, et_p, jnp.zeros((EPAD,), jnp.int32)])
    bounds = (jnp.arange(33, dtype=jnp.int32) * NPT)
    offs = jnp.searchsorted(head_s, bounds).astype(jnp.int32)
    offs = jnp.concatenate([offs, jnp.full((15,), E, jnp.int32)])

    b_pad = jnp.zeros((8, D), jnp.float32).at[0].set(b_fc)

    emb = entity_emb
    for _ in range(2):
        a_tab, b_tab, aug = _prep(emb, relation_emb, W_fc, b_pad)
        acc = _edge_pass(edges_p,
                         a_tab.reshape(N * R), b_tab.reshape(N * R),
                         aug, offs)
        emb = _node_update(acc[:N], emb)
    return emb


# speedup vs baseline: 14.4638x; 1.8984x over previous
import functools

import jax
import jax.numpy as jnp
from jax import lax
from jax.experimental import pallas as pl
from jax.experimental.pallas import tpu as pltpu
from jax.experimental.pallas import tpu_sc as plsc

N = 10000
E = 320000
D = 128
R = 64
NW = 32
NPT = 320
NPAD = NW * NPT
AW = D + 16
G = 128
EPAD = E + 4 * G


def _prep_body(emb_ref, rel_ref, w_ref, b_ref, a_ref, b2_ref, aug_ref):
    rel = rel_ref[...]
    w = w_ref[...]
    p = lax.dot_general(rel, w, (((1,), (0,)), ((), ())),
                        preferred_element_type=jnp.float32)
    emb = emb_ref[...]
    a = lax.dot_general(emb, p[:, :D], (((1,), (1,)), ((), ())),
                        preferred_element_type=jnp.float32)
    bmat = lax.dot_general(emb, p[:, D:], (((1,), (1,)), ((), ())),
                           preferred_element_type=jnp.float32)
    c = lax.dot_general(rel, b_ref[0:1, :], (((1,), (1,)), ((), ())),
                        preferred_element_type=jnp.float32)
    a_ref[...] = a + jnp.reshape(c, (1, R))
    b2_ref[...] = bmat
    aug_ref[...] = jnp.concatenate(
        [emb, jnp.ones((emb.shape[0], 16), jnp.float32)], axis=1)


def _prep(emb, rel, w_fc, b_pad):
    bn = 2000
    grid = N // bn
    return pl.pallas_call(
        _prep_body,
        grid=(grid,),
        in_specs=[
            pl.BlockSpec((bn, D), lambda i: (i, 0)),
            pl.BlockSpec((R, D), lambda i: (0, 0)),
            pl.BlockSpec((D, 2 * D), lambda i: (0, 0)),
            pl.BlockSpec((8, D), lambda i: (0, 0)),
        ],
        out_specs=[
            pl.BlockSpec((bn, R), lambda i: (i, 0)),
            pl.BlockSpec((bn, R), lambda i: (i, 0)),
            pl.BlockSpec((bn, AW), lambda i: (i, 0)),
        ],
        out_shape=[
            jax.ShapeDtypeStruct((N, R), jnp.float32),
            jax.ShapeDtypeStruct((N, R), jnp.float32),
            jax.ShapeDtypeStruct((N, AW), jnp.float32),
        ],
    )(emb, rel, w_fc, b_pad)


def _edge_body(edges_hbm, a_hbm, b_hbm, aug_hbm, offs_hbm, out_hbm,
               edg0, edg1, ia0, ia1, ib0, ib1, mk0, mk1,
               av0, av1, bv0, bv1, rw0, rw1, sv0, sv1, acc,
               offs_sh, sti, stf, offs_sm, hs0, hs1, ss0, ss1,
               sa0, sa1, sb0, sb1, sr0, sr1):
    edg = (edg0, edg1)
    ia = (ia0, ia1)
    ib = (ib0, ib1)
    mk = (mk0, mk1)
    av = (av0, av1)
    bv = (bv0, bv1)
    rw = (rw0, rw1)
    sv = (sv0, sv1)
    hs = (hs0, hs1)
    ss = (ss0, ss1)
    sa = (sa0, sa1)
    sb = (sb0, sb1)
    sr = (sr0, sr1)

    cid = lax.axis_index("c")
    sid = lax.axis_index("s")
    wid = cid * 16 + sid
    n0 = wid * NPT

    def zero_body(i, carry):
        for d in range(AW // 16):
            acc[i, pl.ds(d * 16, 16)] = jnp.zeros((16,), jnp.float32)
        return carry

    lax.fori_loop(0, NPT, zero_body, 0)

    pltpu.sync_copy(offs_hbm, offs_sh.at[sid])
    pltpu.sync_copy(offs_sh.at[sid], offs_sm)
    off_lo = offs_sm[wid]
    off_hi = offs_sm[wid + 1]
    base0 = (off_lo >> 3) << 3
    ngroups = (off_hi - base0 + (G - 1)) >> 7
    npairs = (ngroups + 1) >> 1

    n0v = jnp.full((16,), n0, jnp.int32)
    n1v = jnp.full((16,), n0 + NPT, jnp.int32)
    ev_lim = jnp.full((16,), E, jnp.int32)
    i16 = lax.broadcasted_iota(jnp.int32, (16,), 0)

    def load_prep_fire(g, b):
        base = pl.multiple_of(base0 + g * G, 8)
        pltpu.sync_copy(edges_hbm.at[:, pl.ds(base, G)], edg[b])
        for k8 in range(G // 16):
            sl = pl.ds(k8 * 16, 16)
            hv = edg[b][0, sl]
            tv = edg[b][1, sl]
            ev = edg[b][2, sl]
            ia[b][sl] = hv * R + ev
            ib[b][sl] = tv * R + ev
            eidx = jnp.full((16,), base + k8 * 16, jnp.int32) + i16
            ok = (hv >= n0v) & (hv < n1v) & (eidx < ev_lim)
            mk[b][sl] = jnp.where(ok, jnp.float32(1.0), jnp.float32(0.0))
        ca = pltpu.async_copy(a_hbm.at[ia[b]], av[b], sa[b])
        cb = pltpu.async_copy(b_hbm.at[ib[b]], bv[b], sb[b])
        cr = pltpu.async_copy(aug_hbm.at[edg[b].at[1]], rw[b], sr[b])
        pltpu.sync_copy(edg[b].at[0], sti.at[sid])
        pltpu.sync_copy(sti.at[sid], hs[b])
        return ca, cb, cr

    def wait_set(b):
        pltpu.make_async_copy(a_hbm.at[ia[b]], av[b], sa[b]).wait()
        pltpu.make_async_copy(b_hbm.at[ib[b]], bv[b], sb[b]).wait()
        pltpu.make_async_copy(aug_hbm.at[edg[b].at[1]], rw[b], sr[b]).wait()

    load_prep_fire(0, 0)

    def pair_body(gp, carry):
        for b in range(2):
            g = 2 * gp + b
            load_prep_fire(g + 1, 1 - b)
            wait_set(b)
            for k8 in range(G // 16):
                sl = pl.ds(k8 * 16, 16)
                x = av[b][sl] + bv[b][sl]
                x = jnp.maximum(x, x * jnp.float32(0.2))
                sv[b][sl] = jnp.exp(x) * mk[b][sl]
            pltpu.sync_copy(sv[b], stf.at[sid])
            pltpu.sync_copy(stf.at[sid], ss[b])

            def edge_four(k4, inner):
                ks = [k4 * 4 + u for u in range(4)]
                lids = [jnp.minimum(jnp.maximum(hs[b][k] - n0, 0), NPT - 1)
                        for k in ks]
                sks = [ss[b][k] for k in ks]
                vals = [[rw[b][k, pl.ds(d * 16, 16)] * sks[u]
                         for d in range(AW // 16)]
                        for u, k in enumerate(ks)]
                for u in range(4):
                    for d in range(AW // 16):
                        plsc.addupdate(acc.at[lids[u], pl.ds(d * 16, 16)],
                                       vals[u][d])
                return inner

            lax.fori_loop(0, G // 4, edge_four, 0)
        return carry

    lax.fori_loop(0, npairs, pair_body, 0)
    wait_set(0)
    pltpu.sync_copy(acc, out_hbm.at[pl.ds(n0, NPT)])


_EDGE_SCRATCH = [
    pltpu.VMEM((4, G), jnp.int32),
    pltpu.VMEM((4, G), jnp.int32),
    pltpu.VMEM((G,), jnp.int32),
    pltpu.VMEM((G,), jnp.int32),
    pltpu.VMEM((G,), jnp.int32),
    pltpu.VMEM((G,), jnp.int32),
    pltpu.VMEM((G,), jnp.float32),
    pltpu.VMEM((G,), jnp.float32),
    pltpu.VMEM((G,), jnp.float32),
    pltpu.VMEM((G,), jnp.float32),
    pltpu.VMEM((G,), jnp.float32),
    pltpu.VMEM((G,), jnp.float32),
    pltpu.VMEM((G, AW), jnp.float32),
    pltpu.VMEM((G, AW), jnp.float32),
    pltpu.VMEM((G,), jnp.float32),
    pltpu.VMEM((G,), jnp.float32),
    pltpu.VMEM((NPT, AW), jnp.float32),
    pltpu.VMEM_SHARED((16, 48), jnp.int32),
    pltpu.VMEM_SHARED((16, G), jnp.int32),
    pltpu.VMEM_SHARED((16, G), jnp.float32),
    pltpu.SMEM((48,), jnp.int32),
    pltpu.SMEM((G,), jnp.int32),
    pltpu.SMEM((G,), jnp.int32),
    pltpu.SMEM((G,), jnp.float32),
    pltpu.SMEM((G,), jnp.float32),
    pltpu.SemaphoreType.DMA,
    pltpu.SemaphoreType.DMA,
    pltpu.SemaphoreType.DMA,
    pltpu.SemaphoreType.DMA,
    pltpu.SemaphoreType.DMA,
    pltpu.SemaphoreType.DMA,
]

_edge_pass = functools.partial(
    pl.kernel,
    compiler_params=pltpu.CompilerParams(use_tc_tiling_on_sc=False),
    out_type=jax.ShapeDtypeStruct((NPAD, AW), jnp.float32),
    mesh=plsc.VectorSubcoreMesh(core_axis_name="c", subcore_axis_name="s"),
    scratch_types=_EDGE_SCRATCH,
)(_edge_body)


def _node_body(acc_ref, emb_ref, out_ref):
    agg = acc_ref[:, :D]
    den = acc_ref[:, D:D + 1]
    mid = agg / (den + jnp.float32(1e-16)) + emb_ref[...]
    nrm = jnp.sqrt(jnp.sum(mid * mid, axis=1, keepdims=True))
    out_ref[...] = mid / jnp.maximum(nrm, jnp.float32(1e-12))


def _node_update(acc, emb):
    bn = 2000
    return pl.pallas_call(
        _node_body,
        grid=(N // bn,),
        in_specs=[
            pl.BlockSpec((bn, AW), lambda i: (i, 0)),
            pl.BlockSpec((bn, D), lambda i: (i, 0)),
        ],
        out_specs=pl.BlockSpec((bn, D), lambda i: (i, 0)),
        out_shape=jax.ShapeDtypeStruct((N, D), jnp.float32),
    )(acc, emb)


def kernel(entity_emb, relation_emb, edge_index, edge_type, W_fc, b_fc):
    head = edge_index[0].astype(jnp.int32)
    tail = edge_index[1].astype(jnp.int32)
    et = edge_type.astype(jnp.int32)

    head_s, tail_s, et_s = lax.sort((head, tail, et), dimension=0, num_keys=1, is_stable=False)
    pad = EPAD - E
    head_p = jnp.concatenate([head_s, jnp.zeros((pad,), jnp.int32)])
    tail_p = jnp.concatenate([tail_s, jnp.zeros((pad,), jnp.int32)])
    et_p = jnp.concatenate([et_s, jnp.zeros((pad,), jnp.int32)])
    edges_p = jnp.stack([head_p, tail_p, et_p, jnp.zeros((EPAD,), jnp.int32)])
    bounds = (jnp.arange(33, dtype=jnp.int32) * NPT)
    offs = jnp.searchsorted(head_s, bounds).astype(jnp.int32)
    offs = jnp.concatenate([offs, jnp.full((15,), E, jnp.int32)])

    b_pad = jnp.zeros((8, D), jnp.float32).at[0].set(b_fc)

    emb = entity_emb
    for _ in range(2):
        a_tab, b_tab, aug = _prep(emb, relation_emb, W_fc, b_pad)
        acc = _edge_pass(edges_p,
                         a_tab.reshape(N * R), b_tab.reshape(N * R),
                         aug, offs)
        emb = _node_update(acc[:N], emb)
    return emb
